# baseline (speedup 1.0000x reference)
import functools

import jax
import jax.numpy as jnp
from jax import lax
from jax.experimental import pallas as pl
from jax.experimental.pallas import tpu as pltpu
from jax.experimental.pallas import tpu_sc as plsc

D_MODEL = 64
SEQ = 200
NUM_CORES = 2
NUM_SUBCORES = 16
NUM_WORKERS = NUM_CORES * NUM_SUBCORES

ROWS_PER_CHUNK = 4
CHUNK = ROWS_PER_CHUNK * SEQ

TC_BLK = 2048


def _tc_linearize(table_t, *, vocab):
    grid = (vocab + TC_BLK - 1) // TC_BLK
    sel = jnp.concatenate(
        [jnp.eye(D_MODEL, dtype=jnp.float32),
         jnp.zeros((D_MODEL, 128 - D_MODEL), jnp.float32)],
        axis=1,
    )

    def body(x_ref, sel_ref, o_ref):
        o_ref[...] = lax.dot_general(
            x_ref[...], sel_ref[...],
            (((0,), (0,)), ((), ())),
            precision=lax.Precision.HIGHEST,
        )

    return pl.pallas_call(
        body,
        grid=(grid,),
        in_specs=[
            pl.BlockSpec((D_MODEL, TC_BLK), lambda i: (0, i)),
            pl.BlockSpec((D_MODEL, 128), lambda i: (0, 0)),
        ],
        out_specs=pl.BlockSpec((TC_BLK, 128), lambda i: (i, 0)),
        out_shape=jax.ShapeDtypeStruct((vocab, 128), jnp.float32),
    )(table_t, sel)


def _sc_embed(ids, table, pos_table, *, batch):
    rows_per_worker = batch // NUM_WORKERS
    n_chunks = rows_per_worker // ROWS_PER_CHUNK
    assert n_chunks % 2 == 0

    mesh = plsc.VectorSubcoreMesh(
        core_axis_name="c", subcore_axis_name="s",
        num_cores=NUM_CORES, num_subcores=NUM_SUBCORES,
    )

    @functools.partial(
        pl.kernel,
        mesh=mesh,
        compiler_params=pltpu.CompilerParams(use_tc_tiling_on_sc=False),
        out_type=jax.ShapeDtypeStruct((batch, SEQ, 128), jnp.float32),
        scratch_types=[
            pltpu.VMEM((ROWS_PER_CHUNK, SEQ), jnp.int32),
            pltpu.VMEM((ROWS_PER_CHUNK, SEQ), jnp.int32),
            pltpu.VMEM((CHUNK, D_MODEL), jnp.float32),
            pltpu.VMEM((CHUNK, D_MODEL), jnp.float32),
            pltpu.VMEM((SEQ, D_MODEL), jnp.float32),
            pltpu.SemaphoreType.DMA,
            pltpu.SemaphoreType.DMA,
            pltpu.SemaphoreType.DMA,
            pltpu.SemaphoreType.DMA,
        ],
    )
    def body(ids_hbm, table_hbm, pos_hbm, out_hbm,
             idx0, idx1, rows0, rows1, pos_v, g0, g1, o0, o1):
        idx = (idx0, idx1)
        rows = (rows0, rows1)
        gsem = (g0, g1)
        osem = (o0, o1)
        wid = lax.axis_index("s") * NUM_CORES + lax.axis_index("c")
        pltpu.sync_copy(pos_hbm, pos_v)
        base_row = wid * rows_per_worker

        def stage_idx(b, g):
            brow = base_row + g * ROWS_PER_CHUNK
            pltpu.sync_copy(ids_hbm.at[pl.ds(brow, ROWS_PER_CHUNK)], idx[b])

        def gather_ref(b, j):
            return (
                table_hbm.at[idx[b].at[j]],
                rows[b].at[pl.ds(j * SEQ, SEQ)],
                gsem[b],
            )

        def fire_gathers(b):
            for j in range(ROWS_PER_CHUNK):
                pltpu.async_copy(*gather_ref(b, j))

        def drain_gathers(b):
            for j in range(ROWS_PER_CHUNK):
                pltpu.make_async_copy(*gather_ref(b, j)).wait()

        def store_ref(b, g, j):
            brow = base_row + g * ROWS_PER_CHUNK
            return (
                rows[b].at[pl.ds(j * SEQ, SEQ)],
                out_hbm.at[brow + j, :, pl.ds(0, D_MODEL)],
                osem[b],
            )

        def fire_store(b, g):
            for j in range(ROWS_PER_CHUNK):
                pltpu.async_copy(*store_ref(b, g, j))

        def drain_store(b, g):
            for j in range(ROWS_PER_CHUNK):
                pltpu.make_async_copy(*store_ref(b, g, j)).wait()

        def add_pos(b):
            def add_body(r, inner):
                for c in range(D_MODEL // 16):
                    pv = pos_v[r, pl.ds(c * 16, 16)]
                    for rep in range(ROWS_PER_CHUNK):
                        row = rep * SEQ + r
                        rows[b][row, pl.ds(c * 16, 16)] = (
                            rows[b][row, pl.ds(c * 16, 16)] + pv
                        )
                return inner

            lax.fori_loop(0, SEQ, add_body, 0, unroll=False)

        stage_idx(0, 0)
        fire_gathers(0)
        stage_idx(1, 1)

        def pair_body(gp, carry):
            for b in range(2):
                g = 2 * gp + b
                drain_gathers(b)

                @pl.when(g >= 1)
                def _():
                    drain_store(1 - b, g - 1)

                @pl.when(g + 1 < n_chunks)
                def _():
                    fire_gathers(1 - b)

                add_pos(b)
                fire_store(b, g)

                @pl.when(g + 2 < n_chunks)
                def _():
                    stage_idx(b, g + 2)

            return carry

        lax.fori_loop(0, n_chunks // 2, pair_body, 0, unroll=False)
        drain_store((n_chunks - 1) % 2, n_chunks - 1)

    return body(ids, table, pos_table)


def kernel(token_ids, token_table, pos_table):
    batch, _ = token_ids.shape
    vocab, _ = token_table.shape
    table_t = jnp.transpose(token_table)
    table_lin = _tc_linearize(table_t, vocab=vocab)
    table2 = jnp.reshape(table_lin, (2 * vocab, D_MODEL))
    ids2 = token_ids.astype(jnp.int32) * 2
    o = _sc_embed(ids2, table2, pos_table, batch=batch)
    return lax.slice(o, (0, 0, 0), (batch, SEQ, D_MODEL))


# speedup vs baseline: 1.1530x; 1.1530x over previous
import functools

import jax
import jax.numpy as jnp
from jax import lax
from jax.experimental import pallas as pl
from jax.experimental.pallas import tpu as pltpu
from jax.experimental.pallas import tpu_sc as plsc

D_MODEL = 64
SEQ = 200
NUM_CORES = 2
NUM_SUBCORES = 16
NUM_WORKERS = NUM_CORES * NUM_SUBCORES

ROWS_PER_CHUNK = 4
CHUNK = ROWS_PER_CHUNK * SEQ

TC_BLK = 2048


def _tc_linearize(table_t, *, vocab):
    grid = (vocab + TC_BLK - 1) // TC_BLK
    sel = jnp.concatenate(
        [jnp.eye(D_MODEL, dtype=jnp.float32),
         jnp.zeros((D_MODEL, 128 - D_MODEL), jnp.float32)],
        axis=1,
    )

    def body(x_ref, sel_ref, o_ref):
        o_ref[...] = lax.dot_general(
            x_ref[...], sel_ref[...],
            (((0,), (0,)), ((), ())),
        )

    return pl.pallas_call(
        body,
        grid=(grid,),
        in_specs=[
            pl.BlockSpec((D_MODEL, TC_BLK), lambda i: (0, i)),
            pl.BlockSpec((D_MODEL, 128), lambda i: (0, 0)),
        ],
        out_specs=pl.BlockSpec((TC_BLK, 128), lambda i: (i, 0)),
        out_shape=jax.ShapeDtypeStruct((vocab, 128), jnp.float32),
    )(table_t, sel)


def _sc_embed(ids, table, pos_table, *, batch):
    rows_per_worker = batch // NUM_WORKERS
    n_chunks = rows_per_worker // ROWS_PER_CHUNK
    assert n_chunks % 2 == 0

    mesh = plsc.VectorSubcoreMesh(
        core_axis_name="c", subcore_axis_name="s",
        num_cores=NUM_CORES, num_subcores=NUM_SUBCORES,
    )

    @functools.partial(
        pl.kernel,
        mesh=mesh,
        compiler_params=pltpu.CompilerParams(use_tc_tiling_on_sc=False),
        out_type=jax.ShapeDtypeStruct((batch, SEQ, 128), jnp.float32),
        scratch_types=[
            pltpu.VMEM((ROWS_PER_CHUNK, SEQ), jnp.int32),
            pltpu.VMEM((ROWS_PER_CHUNK, SEQ), jnp.int32),
            pltpu.VMEM((CHUNK, D_MODEL), jnp.float32),
            pltpu.VMEM((CHUNK, D_MODEL), jnp.float32),
            pltpu.VMEM((SEQ, D_MODEL), jnp.float32),
            pltpu.SemaphoreType.DMA,
            pltpu.SemaphoreType.DMA,
            pltpu.SemaphoreType.DMA,
            pltpu.SemaphoreType.DMA,
        ],
    )
    def body(ids_hbm, table_hbm, pos_hbm, out_hbm,
             idx0, idx1, rows0, rows1, pos_v, g0, g1, o0, o1):
        idx = (idx0, idx1)
        rows = (rows0, rows1)
        gsem = (g0, g1)
        osem = (o0, o1)
        wid = lax.axis_index("s") * NUM_CORES + lax.axis_index("c")
        pltpu.sync_copy(pos_hbm, pos_v)
        base_row = wid * rows_per_worker

        def stage_idx(b, g):
            brow = base_row + g * ROWS_PER_CHUNK
            pltpu.sync_copy(ids_hbm.at[pl.ds(brow, ROWS_PER_CHUNK)], idx[b])

        def gather_ref(b, j):
            return (
                table_hbm.at[idx[b].at[j]],
                rows[b].at[pl.ds(j * SEQ, SEQ)],
                gsem[b],
            )

        def fire_gathers(b):
            for j in range(ROWS_PER_CHUNK):
                pltpu.async_copy(*gather_ref(b, j))

        def drain_gathers(b):
            for j in range(ROWS_PER_CHUNK):
                pltpu.make_async_copy(*gather_ref(b, j)).wait()

        def store_ref(b, g, j):
            brow = base_row + g * ROWS_PER_CHUNK
            return (
                rows[b].at[pl.ds(j * SEQ, SEQ)],
                out_hbm.at[brow + j, :, pl.ds(0, D_MODEL)],
                osem[b],
            )

        def fire_store(b, g):
            for j in range(ROWS_PER_CHUNK):
                pltpu.async_copy(*store_ref(b, g, j))

        def drain_store(b, g):
            for j in range(ROWS_PER_CHUNK):
                pltpu.make_async_copy(*store_ref(b, g, j)).wait()

        def add_pos(b):
            def add_body(r, inner):
                for c in range(D_MODEL // 16):
                    pv = pos_v[r, pl.ds(c * 16, 16)]
                    for rep in range(ROWS_PER_CHUNK):
                        row = rep * SEQ + r
                        rows[b][row, pl.ds(c * 16, 16)] = (
                            rows[b][row, pl.ds(c * 16, 16)] + pv
                        )
                return inner

            lax.fori_loop(0, SEQ, add_body, 0, unroll=False)

        stage_idx(0, 0)
        fire_gathers(0)
        stage_idx(1, 1)

        def pair_body(gp, carry):
            for b in range(2):
                g = 2 * gp + b
                drain_gathers(b)

                @pl.when(g >= 1)
                def _():
                    drain_store(1 - b, g - 1)

                @pl.when(g + 1 < n_chunks)
                def _():
                    fire_gathers(1 - b)

                add_pos(b)
                fire_store(b, g)

                @pl.when(g + 2 < n_chunks)
                def _():
                    stage_idx(b, g + 2)

            return carry

        lax.fori_loop(0, n_chunks // 2, pair_body, 0, unroll=False)
        drain_store((n_chunks - 1) % 2, n_chunks - 1)

    return body(ids, table, pos_table)


def kernel(token_ids, token_table, pos_table):
    batch, _ = token_ids.shape
    vocab, _ = token_table.shape
    table_t = jnp.transpose(token_table)
    table_lin = _tc_linearize(table_t, vocab=vocab)
    table2 = jnp.reshape(table_lin, (2 * vocab, D_MODEL))
    ids2 = token_ids.astype(jnp.int32) * 2
    o = _sc_embed(ids2, table2, pos_table, batch=batch)
    return lax.slice(o, (0, 0, 0), (batch, SEQ, D_MODEL))


# speedup vs baseline: 1.5684x; 1.3603x over previous
import functools

import jax
import jax.numpy as jnp
from jax import lax
from jax.experimental import pallas as pl
from jax.experimental.pallas import tpu as pltpu
from jax.experimental.pallas import tpu_sc as plsc

D_MODEL = 64
SEQ = 200
NUM_CORES = 2
NUM_SUBCORES = 16
NUM_WORKERS = NUM_CORES * NUM_SUBCORES

ROWS_PER_CHUNK = 4
CHUNK = ROWS_PER_CHUNK * SEQ

TC_BLK = 8192


def _tc_linearize(table_t, *, vocab):
    grid = (vocab + TC_BLK - 1) // TC_BLK

    def body(x_ref, o_ref):
        y = jnp.transpose(x_ref[...])
        o_ref[...] = lax.pad(
            y, jnp.float32(0), ((0, 0, 0), (0, 128 - D_MODEL, 0))
        )

    return pl.pallas_call(
        body,
        grid=(grid,),
        in_specs=[pl.BlockSpec((D_MODEL, TC_BLK), lambda i: (0, i))],
        out_specs=pl.BlockSpec((TC_BLK, 128), lambda i: (i, 0)),
        out_shape=jax.ShapeDtypeStruct((vocab, 128), jnp.float32),
    )(table_t)


def _sc_embed(ids, table, pos_table, *, batch):
    rows_per_worker = batch // NUM_WORKERS
    n_chunks = rows_per_worker // ROWS_PER_CHUNK
    assert n_chunks % 2 == 0

    mesh = plsc.VectorSubcoreMesh(
        core_axis_name="c", subcore_axis_name="s",
        num_cores=NUM_CORES, num_subcores=NUM_SUBCORES,
    )

    @functools.partial(
        pl.kernel,
        mesh=mesh,
        compiler_params=pltpu.CompilerParams(use_tc_tiling_on_sc=False),
        out_type=jax.ShapeDtypeStruct((batch, SEQ, 128), jnp.float32),
        scratch_types=[
            pltpu.VMEM((ROWS_PER_CHUNK, SEQ), jnp.int32),
            pltpu.VMEM((ROWS_PER_CHUNK, SEQ), jnp.int32),
            pltpu.VMEM((CHUNK, D_MODEL), jnp.float32),
            pltpu.VMEM((CHUNK, D_MODEL), jnp.float32),
            pltpu.VMEM((SEQ, D_MODEL), jnp.float32),
            pltpu.SemaphoreType.DMA,
            pltpu.SemaphoreType.DMA,
            pltpu.SemaphoreType.DMA,
            pltpu.SemaphoreType.DMA,
        ],
    )
    def body(ids_hbm, table_hbm, pos_hbm, out_hbm,
             idx0, idx1, rows0, rows1, pos_v, g0, g1, o0, o1):
        idx = (idx0, idx1)
        rows = (rows0, rows1)
        gsem = (g0, g1)
        osem = (o0, o1)
        wid = lax.axis_index("s") * NUM_CORES + lax.axis_index("c")
        pltpu.sync_copy(pos_hbm, pos_v)
        base_row = wid * rows_per_worker

        def stage_idx(b, g):
            brow = base_row + g * ROWS_PER_CHUNK
            pltpu.sync_copy(ids_hbm.at[pl.ds(brow, ROWS_PER_CHUNK)], idx[b])

        def gather_ref(b, j):
            return (
                table_hbm.at[idx[b].at[j]],
                rows[b].at[pl.ds(j * SEQ, SEQ)],
                gsem[b],
            )

        def fire_gathers(b):
            for j in range(ROWS_PER_CHUNK):
                pltpu.async_copy(*gather_ref(b, j))

        def drain_gathers(b):
            for j in range(ROWS_PER_CHUNK):
                pltpu.make_async_copy(*gather_ref(b, j)).wait()

        def store_ref(b, g, j):
            brow = base_row + g * ROWS_PER_CHUNK
            return (
                rows[b].at[pl.ds(j * SEQ, SEQ)],
                out_hbm.at[brow + j, :, pl.ds(0, D_MODEL)],
                osem[b],
            )

        def fire_store(b, g):
            for j in range(ROWS_PER_CHUNK):
                pltpu.async_copy(*store_ref(b, g, j))

        def drain_store(b, g):
            for j in range(ROWS_PER_CHUNK):
                pltpu.make_async_copy(*store_ref(b, g, j)).wait()

        def add_pos(b):
            def add_body(r, inner):
                for c in range(D_MODEL // 16):
                    pv = pos_v[r, pl.ds(c * 16, 16)]
                    for rep in range(ROWS_PER_CHUNK):
                        row = rep * SEQ + r
                        rows[b][row, pl.ds(c * 16, 16)] = (
                            rows[b][row, pl.ds(c * 16, 16)] + pv
                        )
                return inner

            lax.fori_loop(0, SEQ, add_body, 0, unroll=False)

        stage_idx(0, 0)
        fire_gathers(0)
        stage_idx(1, 1)

        def pair_body(gp, carry):
            for b in range(2):
                g = 2 * gp + b
                drain_gathers(b)

                @pl.when(g >= 1)
                def _():
                    drain_store(1 - b, g - 1)

                @pl.when(g + 1 < n_chunks)
                def _():
                    fire_gathers(1 - b)

                add_pos(b)
                fire_store(b, g)

                @pl.when(g + 2 < n_chunks)
                def _():
                    stage_idx(b, g + 2)

            return carry

        lax.fori_loop(0, n_chunks // 2, pair_body, 0, unroll=False)
        drain_store((n_chunks - 1) % 2, n_chunks - 1)

    return body(ids, table, pos_table)


def kernel(token_ids, token_table, pos_table):
    batch, _ = token_ids.shape
    vocab, _ = token_table.shape
    table_t = jnp.transpose(token_table)
    table_lin = _tc_linearize(table_t, vocab=vocab)
    table2 = jnp.reshape(table_lin, (2 * vocab, D_MODEL))
    ids2 = token_ids.astype(jnp.int32) * 2
    o = _sc_embed(ids2, table2, pos_table, batch=batch)
    return lax.slice(o, (0, 0, 0), (batch, SEQ, D_MODEL))


# speedup vs baseline: 1.6378x; 1.0443x over previous
import functools

import jax
import jax.numpy as jnp
from jax import lax
from jax.experimental import pallas as pl
from jax.experimental.pallas import tpu as pltpu
from jax.experimental.pallas import tpu_sc as plsc

D_MODEL = 64
SEQ = 200
NUM_CORES = 2
NUM_SUBCORES = 16
NUM_WORKERS = NUM_CORES * NUM_SUBCORES

ROWS_PER_CHUNK = 4
CHUNK = ROWS_PER_CHUNK * SEQ

TC_BLK = 32768


def _tc_linearize(table_t, *, vocab):
    grid = (vocab + TC_BLK - 1) // TC_BLK

    def body(x_ref, o_ref):
        y = jnp.transpose(x_ref[...])
        o_ref[...] = lax.pad(
            y, jnp.float32(0), ((0, 0, 0), (0, 128 - D_MODEL, 0))
        )

    return pl.pallas_call(
        body,
        grid=(grid,),
        in_specs=[pl.BlockSpec((D_MODEL, TC_BLK), lambda i: (0, i))],
        out_specs=pl.BlockSpec((TC_BLK, 128), lambda i: (i, 0)),
        out_shape=jax.ShapeDtypeStruct((vocab, 128), jnp.float32),
    )(table_t)


def _sc_embed(ids, table, pos_table, *, batch):
    rows_per_worker = batch // NUM_WORKERS
    n_chunks = rows_per_worker // ROWS_PER_CHUNK
    assert n_chunks % 2 == 0

    mesh = plsc.VectorSubcoreMesh(
        core_axis_name="c", subcore_axis_name="s",
        num_cores=NUM_CORES, num_subcores=NUM_SUBCORES,
    )

    @functools.partial(
        pl.kernel,
        mesh=mesh,
        compiler_params=pltpu.CompilerParams(use_tc_tiling_on_sc=False),
        out_type=jax.ShapeDtypeStruct((batch, SEQ, 128), jnp.float32),
        scratch_types=[
            pltpu.VMEM((ROWS_PER_CHUNK, SEQ), jnp.int32),
            pltpu.VMEM((ROWS_PER_CHUNK, SEQ), jnp.int32),
            pltpu.VMEM((CHUNK, D_MODEL), jnp.float32),
            pltpu.VMEM((CHUNK, D_MODEL), jnp.float32),
            pltpu.VMEM((SEQ, D_MODEL), jnp.float32),
            pltpu.SemaphoreType.DMA,
            pltpu.SemaphoreType.DMA,
            pltpu.SemaphoreType.DMA,
            pltpu.SemaphoreType.DMA,
        ],
    )
    def body(ids_hbm, table_hbm, pos_hbm, out_hbm,
             idx0, idx1, rows0, rows1, pos_v, g0, g1, o0, o1):
        idx = (idx0, idx1)
        rows = (rows0, rows1)
        gsem = (g0, g1)
        osem = (o0, o1)
        wid = lax.axis_index("s") * NUM_CORES + lax.axis_index("c")
        pltpu.sync_copy(pos_hbm, pos_v)
        base_row = wid * rows_per_worker

        def stage_idx(b, g):
            brow = base_row + g * ROWS_PER_CHUNK
            pltpu.sync_copy(ids_hbm.at[pl.ds(brow, ROWS_PER_CHUNK)], idx[b])

        def gather_ref(b, j):
            return (
                table_hbm.at[idx[b].at[j]],
                rows[b].at[pl.ds(j * SEQ, SEQ)],
                gsem[b],
            )

        def fire_gathers(b):
            for j in range(ROWS_PER_CHUNK):
                pltpu.async_copy(*gather_ref(b, j))

        def drain_gathers(b):
            for j in range(ROWS_PER_CHUNK):
                pltpu.make_async_copy(*gather_ref(b, j)).wait()

        def store_ref(b, g, j):
            brow = base_row + g * ROWS_PER_CHUNK
            return (
                rows[b].at[pl.ds(j * SEQ, SEQ)],
                out_hbm.at[brow + j, :, pl.ds(0, D_MODEL)],
                osem[b],
            )

        def fire_store(b, g):
            for j in range(ROWS_PER_CHUNK):
                pltpu.async_copy(*store_ref(b, g, j))

        def drain_store(b, g):
            for j in range(ROWS_PER_CHUNK):
                pltpu.make_async_copy(*store_ref(b, g, j)).wait()

        def add_pos(b):
            def add_body(r, inner):
                for c in range(D_MODEL // 16):
                    pv = pos_v[r, pl.ds(c * 16, 16)]
                    for rep in range(ROWS_PER_CHUNK):
                        row = rep * SEQ + r
                        rows[b][row, pl.ds(c * 16, 16)] = (
                            rows[b][row, pl.ds(c * 16, 16)] + pv
                        )
                return inner

            lax.fori_loop(0, SEQ, add_body, 0, unroll=False)

        stage_idx(0, 0)
        fire_gathers(0)
        stage_idx(1, 1)

        def pair_body(gp, carry):
            for b in range(2):
                g = 2 * gp + b
                drain_gathers(b)

                @pl.when(g >= 1)
                def _():
                    drain_store(1 - b, g - 1)

                @pl.when(g + 1 < n_chunks)
                def _():
                    fire_gathers(1 - b)

                add_pos(b)
                fire_store(b, g)

                @pl.when(g + 2 < n_chunks)
                def _():
                    stage_idx(b, g + 2)

            return carry

        lax.fori_loop(0, n_chunks // 2, pair_body, 0, unroll=False)
        drain_store((n_chunks - 1) % 2, n_chunks - 1)

    return body(ids, table, pos_table)


def kernel(token_ids, token_table, pos_table):
    batch, _ = token_ids.shape
    vocab, _ = token_table.shape
    table_t = jnp.transpose(token_table)
    table_lin = _tc_linearize(table_t, vocab=vocab)
    table2 = jnp.reshape(table_lin, (2 * vocab, D_MODEL))
    ids2 = token_ids.astype(jnp.int32) * 2
    o = _sc_embed(ids2, table2, pos_table, batch=batch)
    return lax.slice(o, (0, 0, 0), (batch, SEQ, D_MODEL))
